# baseline (speedup 1.0000x reference)
import jax
import jax.numpy as jnp
from jax import lax
from jax.experimental import pallas as pl
from jax.experimental.pallas import tpu as pltpu
from jax.experimental.pallas import tpu_sc as plsc

_EMBED = 32
_MAXLEN = 200
_NC = 2
_NS = 16
_NW = _NC * _NS
_CH = 128
_NBUF = 6
_DEPTH = 4

_VOCAB = 1000000
_NBLK = _VOCAB // 128
_TAIL = _VOCAB - _NBLK * 128


def _transpose_body(tokT_hbm, tail_hbm, out_hbm, tbuf, obuf, tailv, sem, osem):
    wid = lax.axis_index("s") * _NC + lax.axis_index("c")
    per_w = _NBLK // _NW
    extra = _NBLK - per_w * _NW
    start = wid * per_w + lax.min(wid, extra)
    count = per_w + jnp.where(wid < extra, 1, 0)

    ci = lax.iota(jnp.int32, 16)
    cb_lo, s_lo = ci >> 3, ci & 7
    cb_hi, s_hi = (ci + 16) >> 3, (ci + 16) & 7

    def blk(i, carry):
        b = start + i
        for cb in range(4):
            pltpu.async_copy(
                tokT_hbm.at[pl.ds(cb * 8, 8), pl.ds(b * 128, 128)],
                tbuf.at[cb, pl.ds(0, 8), pl.ds(0, 128)], sem)
        for cb in range(4):
            pltpu.make_async_copy(
                tokT_hbm.at[pl.ds(0, 8), pl.ds(0, 128)],
                tbuf.at[cb, pl.ds(0, 8), pl.ds(0, 128)], sem).wait()

        def tok(t, c2):
            tv = jnp.full((16,), 0, jnp.int32) + t
            v0 = plsc.load_gather(tbuf, [cb_lo, s_lo, tv])
            v1 = plsc.load_gather(tbuf, [cb_hi, s_hi, tv])
            obuf[pl.ds(t * _EMBED, 16)] = v0
            obuf[pl.ds(t * _EMBED + 16, 16)] = v1
            return c2

        lax.fori_loop(0, 128, tok, 0, unroll=4)

        pltpu.async_copy(obuf, out_hbm.at[pl.ds(b * 4096, 4096)], osem)
        pltpu.make_async_copy(obuf, out_hbm.at[pl.ds(0, 4096)], osem).wait()
        return carry

    lax.fori_loop(0, count, blk, 0)

    @pl.when(wid == 0)
    def _():
        pltpu.sync_copy(tail_hbm, tailv)
        pltpu.sync_copy(tailv, out_hbm.at[pl.ds(_NBLK * 128 * _EMBED,
                                                _TAIL * _EMBED)])


def _gather_body(x_hbm, tok_hbm, pos_hbm, out_hbm, idx_v, pos2_v, buf,
                 gsem, osem):
    n_chunks_total = x_hbm.shape[0]
    nch = n_chunks_total // _NW
    rpw = nch * _CH
    wid = lax.axis_index("s") * _NC + lax.axis_index("c")
    wbase = wid * rpw

    pltpu.sync_copy(x_hbm.at[pl.ds(wid * nch, nch)], idx_v)
    pltpu.sync_copy(pos_hbm, pos2_v.at[pl.ds(0, _MAXLEN)])
    pltpu.sync_copy(pos_hbm, pos2_v.at[pl.ds(_MAXLEN, _MAXLEN)])

    for p in range(_DEPTH):
        pltpu.async_copy(tok_hbm.at[idx_v.at[p]], buf.at[p], gsem)

    def chunk_body(j, carry):
        b = lax.rem(j, _NBUF)
        pltpu.make_async_copy(tok_hbm.at[idx_v.at[0]], buf.at[0], gsem).wait()

        poff = lax.rem(j * _CH, _MAXLEN)

        def row_body(r, c):
            plsc.addupdate(buf.at[b, r, pl.ds(0, 16)],
                           pos2_v[poff + r, pl.ds(0, 16)])
            plsc.addupdate(buf.at[b, r, pl.ds(16, 16)],
                           pos2_v[poff + r, pl.ds(16, 16)])
            return c

        lax.fori_loop(0, _CH, row_body, 0, unroll=4)

        pltpu.async_copy(buf.at[b], out_hbm.at[pl.ds(wbase + j * _CH, _CH)],
                         osem)

        @pl.when(j >= _NBUF - _DEPTH)
        def _():
            pltpu.make_async_copy(buf.at[0], out_hbm.at[pl.ds(wbase, _CH)],
                                  osem).wait()

        @pl.when(j + _DEPTH < nch)
        def _():
            pltpu.async_copy(tok_hbm.at[idx_v.at[j + _DEPTH]],
                             buf.at[lax.rem(j + _DEPTH, _NBUF)], gsem)

        return carry

    lax.fori_loop(0, nch, chunk_body, 0)

    pltpu.make_async_copy(buf.at[0], out_hbm.at[pl.ds(wbase, _CH)], osem).wait()
    pltpu.make_async_copy(buf.at[0], out_hbm.at[pl.ds(wbase, _CH)], osem).wait()


@jax.jit
def _sc_embed(x, token_table, pos_table):
    batch, maxlen = x.shape
    mesh = plsc.VectorSubcoreMesh(core_axis_name="c", subcore_axis_name="s")

    tokT = token_table.T
    tail = token_table[_NBLK * 128:].reshape(-1)

    relaid = pl.kernel(
        _transpose_body,
        out_type=jax.ShapeDtypeStruct((_VOCAB * _EMBED,), jnp.float32),
        mesh=mesh,
        scratch_types=[
            pltpu.VMEM((4, 8, 129), jnp.float32),
            pltpu.VMEM((4096,), jnp.float32),
            pltpu.VMEM((_TAIL * _EMBED,), jnp.float32),
            pltpu.SemaphoreType.DMA,
            pltpu.SemaphoreType.DMA,
        ],
        compiler_params=pltpu.CompilerParams(use_tc_tiling_on_sc=True,
                                             needs_layout_passes=False),
    )(tokT, tail)

    tok_lin = relaid.reshape(_VOCAB, _EMBED)
    x_idx = x.astype(jnp.int32).reshape(batch * maxlen // _CH, _CH)
    nch = x_idx.shape[0] // _NW

    out = pl.kernel(
        _gather_body,
        out_type=jax.ShapeDtypeStruct((batch * maxlen, _EMBED), jnp.float32),
        mesh=mesh,
        scratch_types=[
            pltpu.VMEM((nch, _CH), jnp.int32),
            pltpu.VMEM((2 * _MAXLEN, _EMBED), jnp.float32),
            pltpu.VMEM((_NBUF, _CH, _EMBED), jnp.float32),
            pltpu.SemaphoreType.DMA,
            pltpu.SemaphoreType.DMA,
        ],
        compiler_params=pltpu.CompilerParams(use_tc_tiling_on_sc=False),
    )(x_idx, tok_lin, pos_table)
    return out.reshape(batch, maxlen, _EMBED)


def kernel(x, token_table, pos_table):
    return _sc_embed(x, token_table, pos_table)


# speedup vs baseline: 1.2417x; 1.2417x over previous
import jax
import jax.numpy as jnp
from jax import lax
from jax.experimental import pallas as pl
from jax.experimental.pallas import tpu as pltpu
from jax.experimental.pallas import tpu_sc as plsc

_EMBED = 32
_MAXLEN = 200
_NC = 2
_NS = 16
_NW = _NC * _NS
_CH = 128
_NBUF = 6
_DEPTH = 4

_VOCAB = 1000000
_NBLK = _VOCAB // 128
_TAIL = _VOCAB - _NBLK * 128


def _transpose_body(tokT_hbm, tail_hbm, out_hbm, tbuf, obuf, tailv, sem, osem):
    wid = lax.axis_index("s") * _NC + lax.axis_index("c")
    per_w = _NBLK // _NW
    extra = _NBLK - per_w * _NW
    start = wid * per_w + lax.min(wid, extra)
    count = per_w + jnp.where(wid < extra, 1, 0)

    ci = lax.iota(jnp.int32, 16)
    cb_lo, s_lo = ci >> 3, ci & 7
    cb_hi, s_hi = (ci + 16) >> 3, (ci + 16) & 7

    def fire(b, p):
        for cb in range(4):
            pltpu.async_copy(
                tokT_hbm.at[pl.ds(cb * 8, 8), pl.ds(b * 128, 128)],
                tbuf.at[p, cb, pl.ds(0, 8), pl.ds(0, 128)], sem)

    fire(start, 0)

    def blk(i, carry):
        p = lax.rem(i, 2)

        @pl.when(i + 1 < count)
        def _():
            fire(start + i + 1, lax.rem(i + 1, 2))

        for cb in range(4):
            pltpu.make_async_copy(
                tokT_hbm.at[pl.ds(0, 8), pl.ds(0, 128)],
                tbuf.at[0, cb, pl.ds(0, 8), pl.ds(0, 128)], sem).wait()

        @pl.when(i >= 2)
        def _():
            pltpu.make_async_copy(obuf.at[0],
                                  out_hbm.at[pl.ds(0, 4096)], osem).wait()

        def tok(t, c2):
            tv = jnp.full((16,), 0, jnp.int32) + t
            v0 = plsc.load_gather(tbuf.at[p], [cb_lo, s_lo, tv])
            v1 = plsc.load_gather(tbuf.at[p], [cb_hi, s_hi, tv])
            obuf[p, pl.ds(t * _EMBED, 16)] = v0
            obuf[p, pl.ds(t * _EMBED + 16, 16)] = v1
            return c2

        lax.fori_loop(0, 128, tok, 0, unroll=4)

        pltpu.async_copy(obuf.at[p],
                         out_hbm.at[pl.ds((start + i) * 4096, 4096)], osem)
        return carry

    lax.fori_loop(0, count, blk, 0)

    pltpu.make_async_copy(obuf.at[0], out_hbm.at[pl.ds(0, 4096)], osem).wait()
    pltpu.make_async_copy(obuf.at[0], out_hbm.at[pl.ds(0, 4096)], osem).wait()

    @pl.when(wid == 0)
    def _():
        pltpu.sync_copy(tail_hbm, tailv)
        pltpu.sync_copy(tailv, out_hbm.at[pl.ds(_NBLK * 128 * _EMBED,
                                                _TAIL * _EMBED)])


def _gather_body(x_hbm, tok_hbm, pos_hbm, out_hbm, idx_v, pos2_v, buf,
                 gsem, osem):
    n_chunks_total = x_hbm.shape[0]
    nch = n_chunks_total // _NW
    rpw = nch * _CH
    wid = lax.axis_index("s") * _NC + lax.axis_index("c")
    wbase = wid * rpw

    pltpu.sync_copy(x_hbm.at[pl.ds(wid * nch, nch)], idx_v)
    pltpu.sync_copy(pos_hbm, pos2_v.at[pl.ds(0, _MAXLEN)])
    pltpu.sync_copy(pos_hbm, pos2_v.at[pl.ds(_MAXLEN, _MAXLEN)])

    for p in range(_DEPTH):
        pltpu.async_copy(tok_hbm.at[idx_v.at[p]], buf.at[p], gsem)

    def chunk_body(j, carry):
        b = lax.rem(j, _NBUF)
        pltpu.make_async_copy(tok_hbm.at[idx_v.at[0]], buf.at[0], gsem).wait()

        poff = lax.rem(j * _CH, _MAXLEN)

        def row_body(r, c):
            plsc.addupdate(buf.at[b, r, pl.ds(0, 16)],
                           pos2_v[poff + r, pl.ds(0, 16)])
            plsc.addupdate(buf.at[b, r, pl.ds(16, 16)],
                           pos2_v[poff + r, pl.ds(16, 16)])
            return c

        lax.fori_loop(0, _CH, row_body, 0, unroll=4)

        pltpu.async_copy(buf.at[b], out_hbm.at[pl.ds(wbase + j * _CH, _CH)],
                         osem)

        @pl.when(j >= _NBUF - _DEPTH)
        def _():
            pltpu.make_async_copy(buf.at[0], out_hbm.at[pl.ds(wbase, _CH)],
                                  osem).wait()

        @pl.when(j + _DEPTH < nch)
        def _():
            pltpu.async_copy(tok_hbm.at[idx_v.at[j + _DEPTH]],
                             buf.at[lax.rem(j + _DEPTH, _NBUF)], gsem)

        return carry

    lax.fori_loop(0, nch, chunk_body, 0)

    pltpu.make_async_copy(buf.at[0], out_hbm.at[pl.ds(wbase, _CH)], osem).wait()
    pltpu.make_async_copy(buf.at[0], out_hbm.at[pl.ds(wbase, _CH)], osem).wait()


@jax.jit
def _sc_embed(x, token_table, pos_table):
    batch, maxlen = x.shape
    mesh = plsc.VectorSubcoreMesh(core_axis_name="c", subcore_axis_name="s")

    tokT = token_table.T
    tail = token_table[_NBLK * 128:].reshape(-1)

    relaid = pl.kernel(
        _transpose_body,
        out_type=jax.ShapeDtypeStruct((_VOCAB * _EMBED,), jnp.float32),
        mesh=mesh,
        scratch_types=[
            pltpu.VMEM((2, 4, 8, 129), jnp.float32),
            pltpu.VMEM((2, 4096), jnp.float32),
            pltpu.VMEM((_TAIL * _EMBED,), jnp.float32),
            pltpu.SemaphoreType.DMA,
            pltpu.SemaphoreType.DMA,
        ],
        compiler_params=pltpu.CompilerParams(use_tc_tiling_on_sc=True,
                                             needs_layout_passes=False),
    )(tokT, tail)

    tok_lin = relaid.reshape(_VOCAB, _EMBED)
    x_idx = x.astype(jnp.int32).reshape(batch * maxlen // _CH, _CH)
    nch = x_idx.shape[0] // _NW

    out = pl.kernel(
        _gather_body,
        out_type=jax.ShapeDtypeStruct((batch * maxlen, _EMBED), jnp.float32),
        mesh=mesh,
        scratch_types=[
            pltpu.VMEM((nch, _CH), jnp.int32),
            pltpu.VMEM((2 * _MAXLEN, _EMBED), jnp.float32),
            pltpu.VMEM((_NBUF, _CH, _EMBED), jnp.float32),
            pltpu.SemaphoreType.DMA,
            pltpu.SemaphoreType.DMA,
        ],
        compiler_params=pltpu.CompilerParams(use_tc_tiling_on_sc=False),
    )(x_idx, tok_lin, pos_table)
    return out.reshape(batch, maxlen, _EMBED)


def kernel(x, token_table, pos_table):
    return _sc_embed(x, token_table, pos_table)


# speedup vs baseline: 1.6546x; 1.3326x over previous
import jax
import jax.numpy as jnp
from jax import lax
from jax.experimental import pallas as pl
from jax.experimental.pallas import tpu as pltpu
from jax.experimental.pallas import tpu_sc as plsc

_EMBED = 32
_MAXLEN = 200
_NC = 2
_NS = 16
_NW = _NC * _NS
_CH = 128
_NBUF = 6
_DEPTH = 4


def _gather_body(x_hbm, tok_hbm, pos_hbm, out_hbm, idx_v, pos2_v, buf,
                 gsem, osem):
    n_chunks_total = x_hbm.shape[0]
    nch = n_chunks_total // _NW
    rpw = nch * _CH
    wid = lax.axis_index("s") * _NC + lax.axis_index("c")
    wbase = wid * rpw

    pltpu.sync_copy(x_hbm.at[pl.ds(wid * nch, nch)], idx_v)
    pltpu.sync_copy(pos_hbm, pos2_v.at[pl.ds(0, _MAXLEN)])
    pltpu.sync_copy(pos_hbm, pos2_v.at[pl.ds(_MAXLEN, _MAXLEN)])

    for p in range(_DEPTH):
        pltpu.async_copy(tok_hbm.at[idx_v.at[p]], buf.at[p], gsem)

    def chunk_body(j, carry):
        b = lax.rem(j, _NBUF)
        pltpu.make_async_copy(tok_hbm.at[idx_v.at[0]], buf.at[0], gsem).wait()

        poff = lax.rem(j * _CH, _MAXLEN)

        def row_body(r, c):
            plsc.addupdate(buf.at[b, r, pl.ds(0, 16)],
                           pos2_v[poff + r, pl.ds(0, 16)])
            plsc.addupdate(buf.at[b, r, pl.ds(16, 16)],
                           pos2_v[poff + r, pl.ds(16, 16)])
            return c

        lax.fori_loop(0, _CH, row_body, 0, unroll=4)

        pltpu.async_copy(buf.at[b], out_hbm.at[pl.ds(wbase + j * _CH, _CH)],
                         osem)

        @pl.when(j >= _NBUF - _DEPTH)
        def _():
            pltpu.make_async_copy(buf.at[0], out_hbm.at[pl.ds(wbase, _CH)],
                                  osem).wait()

        @pl.when(j + _DEPTH < nch)
        def _():
            pltpu.async_copy(tok_hbm.at[idx_v.at[j + _DEPTH]],
                             buf.at[lax.rem(j + _DEPTH, _NBUF)], gsem)

        return carry

    lax.fori_loop(0, nch, chunk_body, 0)

    pltpu.make_async_copy(buf.at[0], out_hbm.at[pl.ds(wbase, _CH)], osem).wait()
    pltpu.make_async_copy(buf.at[0], out_hbm.at[pl.ds(wbase, _CH)], osem).wait()


@jax.jit
def _sc_embed(x, token_table, pos_table):
    batch, maxlen = x.shape
    mesh = plsc.VectorSubcoreMesh(core_axis_name="c", subcore_axis_name="s")

    tok_lin = token_table
    x_idx = x.astype(jnp.int32).reshape(batch * maxlen // _CH, _CH)
    nch = x_idx.shape[0] // _NW

    out = pl.kernel(
        _gather_body,
        out_type=jax.ShapeDtypeStruct((batch * maxlen, _EMBED), jnp.float32),
        mesh=mesh,
        scratch_types=[
            pltpu.VMEM((nch, _CH), jnp.int32),
            pltpu.VMEM((2 * _MAXLEN, _EMBED), jnp.float32),
            pltpu.VMEM((_NBUF, _CH, _EMBED), jnp.float32),
            pltpu.SemaphoreType.DMA,
            pltpu.SemaphoreType.DMA,
        ],
        compiler_params=pltpu.CompilerParams(use_tc_tiling_on_sc=False),
    )(x_idx, tok_lin, pos_table)
    return out.reshape(batch, maxlen, _EMBED)


def kernel(x, token_table, pos_table):
    return _sc_embed(x, token_table, pos_table)
